# baseline (speedup 1.0000x reference)
import jax
import jax.numpy as jnp
from jax.experimental import pallas as pl
from jax.experimental.pallas import tpu as pltpu
import functools

N = 8192
M = 8192
BJ = 512
NJ = M // BJ


def _chamfer_body(pred_ref, tgt_ref, out_ref, rowmin_ref, colsum_ref):
    j = pl.program_id(0)

    p = pred_ref[:]
    t = tgt_ref[:]
    g = jax.lax.dot_general(
        p, t, (((1,), (0,)), ((), ())),
        preferred_element_type=jnp.float32,
        precision=jax.lax.Precision.HIGHEST,
    )
    pn = jnp.sum(p * p, axis=1, keepdims=True)
    tn = jnp.sum(t * t, axis=0, keepdims=True)
    d2 = (pn + tn) - 2.0 * g

    block_rowmin = jnp.maximum(jnp.min(d2, axis=1, keepdims=True), 0.0)
    colmin = jnp.maximum(jnp.min(d2, axis=0, keepdims=True), 0.0)

    @pl.when(j == 0)
    def _init():
        rowmin_ref[:, :] = block_rowmin
        colsum_ref[0, 0] = 0.0

    @pl.when(j > 0)
    def _acc():
        rowmin_ref[:, :] = jnp.minimum(rowmin_ref[:, :], block_rowmin)

    colsum_ref[0, 0] += jnp.sum(jnp.sqrt(colmin))

    @pl.when(j == NJ - 1)
    def _final():
        rowsum = jnp.sum(jnp.sqrt(rowmin_ref[:, :]))
        out_ref[0, 0] = rowsum * (1.0 / N) + colsum_ref[0, 0] * (1.0 / M)


@jax.jit
def kernel(pred, target):
    tgt_t = target.T
    out = pl.pallas_call(
        _chamfer_body,
        grid=(NJ,),
        in_specs=[
            pl.BlockSpec((N, 3), lambda j: (0, 0)),
            pl.BlockSpec((3, BJ), lambda j: (0, j)),
        ],
        out_specs=pl.BlockSpec((1, 1), lambda j: (0, 0), memory_space=pltpu.SMEM),
        out_shape=jax.ShapeDtypeStruct((1, 1), jnp.float32),
        scratch_shapes=[
            pltpu.VMEM((N, 1), jnp.float32),
            pltpu.SMEM((1, 1), jnp.float32),
        ],
    )(pred, tgt_t)
    return out[0, 0]


# speedup vs baseline: 2.2959x; 2.2959x over previous
import jax
import jax.numpy as jnp
from jax.experimental import pallas as pl
from jax.experimental.pallas import tpu as pltpu
import functools

N = 8192
M = 8192
BJ = 512
NJ = M // BJ


def _chamfer_body(pred_ref, tgt_ref, out_ref, rowmin_ref, colsum_ref):
    j = pl.program_id(0)

    px = pred_ref[:, 0:1]
    py = pred_ref[:, 1:2]
    pz = pred_ref[:, 2:3]
    tx = tgt_ref[0:1, :]
    ty = tgt_ref[1:2, :]
    tz = tgt_ref[2:3, :]

    dx = px - tx
    dy = py - ty
    dz = pz - tz
    d2 = dz * dz + (dy * dy + dx * dx)

    folded = jnp.minimum(
        jnp.minimum(d2[:, 0:128], d2[:, 128:256]),
        jnp.minimum(d2[:, 256:384], d2[:, 384:512]),
    )
    colmin = jnp.min(d2, axis=0, keepdims=True)

    @pl.when(j == 0)
    def _init():
        rowmin_ref[:, :] = folded
        colsum_ref[0, 0] = 0.0

    @pl.when(j > 0)
    def _acc():
        rowmin_ref[:, :] = jnp.minimum(rowmin_ref[:, :], folded)

    colsum_ref[0, 0] += jnp.sum(jnp.sqrt(colmin))

    @pl.when(j == NJ - 1)
    def _final():
        rowmin = jnp.min(rowmin_ref[:, :], axis=1)
        rowsum = jnp.sum(jnp.sqrt(rowmin))
        out_ref[0, 0] = rowsum * (1.0 / N) + colsum_ref[0, 0] * (1.0 / M)


@jax.jit
def kernel(pred, target):
    tgt_t = target.T
    out = pl.pallas_call(
        _chamfer_body,
        grid=(NJ,),
        in_specs=[
            pl.BlockSpec((N, 3), lambda j: (0, 0)),
            pl.BlockSpec((3, BJ), lambda j: (0, j)),
        ],
        out_specs=pl.BlockSpec((1, 1), lambda j: (0, 0), memory_space=pltpu.SMEM),
        out_shape=jax.ShapeDtypeStruct((1, 1), jnp.float32),
        scratch_shapes=[
            pltpu.VMEM((N, 128), jnp.float32),
            pltpu.SMEM((1, 1), jnp.float32),
        ],
    )(pred, tgt_t)
    return out[0, 0]


# speedup vs baseline: 2.3321x; 1.0158x over previous
import jax
import jax.numpy as jnp
from jax.experimental import pallas as pl
from jax.experimental.pallas import tpu as pltpu
import functools

N = 8192
M = 8192
BJ = 512
NJ = M // BJ


def _chamfer_body(pred_ref, tgt_ref, out_ref, rowmin_ref, colmin_ref):
    j = pl.program_id(0)

    px = pred_ref[:, 0:1]
    py = pred_ref[:, 1:2]
    pz = pred_ref[:, 2:3]
    tx = tgt_ref[0:1, :]
    ty = tgt_ref[1:2, :]
    tz = tgt_ref[2:3, :]

    dx = px - tx
    dy = py - ty
    dz = pz - tz
    d2 = dz * dz + (dy * dy + dx * dx)

    folded = jnp.minimum(
        jnp.minimum(d2[:, 0:128], d2[:, 128:256]),
        jnp.minimum(d2[:, 256:384], d2[:, 384:512]),
    )
    colmin = jnp.min(d2, axis=0, keepdims=True)

    @pl.when(j == 0)
    def _init():
        rowmin_ref[:, :] = folded

    @pl.when(j > 0)
    def _acc():
        rowmin_ref[:, :] = jnp.minimum(rowmin_ref[:, :], folded)

    colmin_ref[:, pl.ds(j * BJ, BJ)] = colmin

    @pl.when(j == NJ - 1)
    def _final():
        rowmin = jnp.min(rowmin_ref[:, :], axis=1)
        rowsum = jnp.sum(jnp.sqrt(rowmin))
        colsum = jnp.sum(jnp.sqrt(colmin_ref[:, :]))
        out_ref[0, 0] = rowsum * (1.0 / N) + colsum * (1.0 / M)


@jax.jit
def kernel(pred, target):
    tgt_t = target.T
    out = pl.pallas_call(
        _chamfer_body,
        grid=(NJ,),
        in_specs=[
            pl.BlockSpec((N, 3), lambda j: (0, 0)),
            pl.BlockSpec((3, BJ), lambda j: (0, j)),
        ],
        out_specs=pl.BlockSpec((1, 1), lambda j: (0, 0), memory_space=pltpu.SMEM),
        out_shape=jax.ShapeDtypeStruct((1, 1), jnp.float32),
        scratch_shapes=[
            pltpu.VMEM((N, 128), jnp.float32),
            pltpu.VMEM((1, M), jnp.float32),
        ],
    )(pred, tgt_t)
    return out[0, 0]
